# baseline (speedup 1.0000x reference)
import functools

import jax
import jax.numpy as jnp
from jax.experimental import pallas as pl
from jax.experimental.pallas import tpu as pltpu

NEG_SLOPE = 0.5
W = 16640


def _leaky(x):
    return jnp.where(x >= 0, x, NEG_SLOPE * x)


def _dot(x, y):
    return jax.lax.dot_general(
        x, y, (((1,), (0,)), ((), ())), preferred_element_type=jnp.float32
    )


def _hgnn_body(adj_ref, embT_ref, outT_ref, adj_sc, latT_sc, lat_sc,
               *, nblk, rem, remw, h, d):
    p = pl.program_id(0)
    i = pl.program_id(1)

    @pl.when(p == 0)
    def _phase0():
        ab = adj_ref[...].astype(jnp.bfloat16)
        e = embT_ref[...].astype(jnp.bfloat16)

        @pl.when(i < nblk - 1)
        def _full():
            adj_sc[:, pl.ds(i * W, W)] = jnp.swapaxes(ab, 0, 1)
            part = _dot(e, ab)

            @pl.when(i == 0)
            def _():
                latT_sc[...] = part

            @pl.when(i > 0)
            def _():
                latT_sc[...] += part

        @pl.when(i == nblk - 1)
        def _partial():
            rowmask = jax.lax.broadcasted_iota(jnp.int32, (W, h), 0) < rem
            ab2 = jnp.where(rowmask, ab, jnp.bfloat16(0))
            adj_sc[:, pl.ds(i * W, remw)] = jnp.swapaxes(ab2, 0, 1)[:, :remw]
            lanemask = jax.lax.broadcasted_iota(jnp.int32, (d, W), 1) < rem
            e2 = jnp.where(lanemask, e, jnp.bfloat16(0))
            latT_sc[...] += _dot(e2, ab2)

    @pl.when(p == 1)
    def _phase1():
        @pl.when(i == 0)
        def _():
            lat_sc[...] = _leaky(latT_sc[...]).astype(jnp.bfloat16)

        j = nblk - 1 - i

        @pl.when(i == 0)
        def _last_block():
            outT_ref[:, :remw] = _leaky(
                _dot(lat_sc[...], adj_sc[:, pl.ds(j * W, remw)])
            )

        @pl.when(i > 0)
        def _full_block():
            outT_ref[...] = _leaky(
                _dot(lat_sc[...], adj_sc[:, pl.ds(j * W, W)])
            )


@jax.jit
def kernel(adj, embeds):
    n, h = adj.shape
    d = embeds.shape[1]
    nblk = -(-n // W)
    rem = n - (nblk - 1) * W
    n128 = -(-n // 128) * 128
    remw = min(W, n128 - (nblk - 1) * W)

    embT = embeds.T
    body = functools.partial(
        _hgnn_body, nblk=nblk, rem=rem, remw=remw, h=h, d=d
    )
    retT = pl.pallas_call(
        body,
        grid=(2, nblk),
        in_specs=[
            pl.BlockSpec((W, h),
                         lambda p, i: (i * (1 - p) + (nblk - 1) * p, 0)),
            pl.BlockSpec((d, W),
                         lambda p, i: (0, i * (1 - p) + (nblk - 1) * p)),
        ],
        out_specs=pl.BlockSpec((d, W), lambda p, i: (0, nblk - 1 - i * p)),
        out_shape=jax.ShapeDtypeStruct((d, n), jnp.float32),
        scratch_shapes=[
            pltpu.VMEM((h, (nblk - 1) * W + remw), jnp.bfloat16),
            pltpu.VMEM((d, h), jnp.float32),
            pltpu.VMEM((d, h), jnp.bfloat16),
        ],
    )(adj, embT)
    return retT.T


# speedup vs baseline: 1.0399x; 1.0399x over previous
import functools

import jax
import jax.numpy as jnp
from jax.experimental import pallas as pl
from jax.experimental.pallas import tpu as pltpu

NEG_SLOPE = 0.5
W = 12800


def _leaky(x):
    return jnp.where(x >= 0, x, NEG_SLOPE * x)


def _dot(x, y):
    return jax.lax.dot_general(
        x, y, (((1,), (0,)), ((), ())), preferred_element_type=jnp.float32
    )


def _hgnn_body(adj_ref, embT_ref, outT_ref, adj_sc, latT_sc, lat_sc,
               *, nblk, rem, remw, h, d):
    p = pl.program_id(0)
    i = pl.program_id(1)

    @pl.when(p == 0)
    def _phase0():
        ab = adj_ref[...].astype(jnp.bfloat16)
        e = embT_ref[...].astype(jnp.bfloat16)

        @pl.when(i < nblk - 1)
        def _full():
            adj_sc[:, pl.ds(i * W, W)] = jnp.swapaxes(ab, 0, 1)
            part = _dot(e, ab)

            @pl.when(i == 0)
            def _():
                latT_sc[...] = part

            @pl.when(i > 0)
            def _():
                latT_sc[...] += part

        @pl.when(i == nblk - 1)
        def _partial():
            rowmask = jax.lax.broadcasted_iota(jnp.int32, (W, h), 0) < rem
            ab2 = jnp.where(rowmask, ab, jnp.bfloat16(0))
            adj_sc[:, pl.ds(i * W, remw)] = jnp.swapaxes(ab2, 0, 1)[:, :remw]
            lanemask = jax.lax.broadcasted_iota(jnp.int32, (d, W), 1) < rem
            e2 = jnp.where(lanemask, e, jnp.bfloat16(0))
            latT_sc[...] += _dot(e2, ab2)

    @pl.when(p == 1)
    def _phase1():
        @pl.when(i == 0)
        def _():
            lat_sc[...] = _leaky(latT_sc[...]).astype(jnp.bfloat16)

        j = nblk - 1 - i

        @pl.when(i == 0)
        def _last_block():
            outT_ref[:, :remw] = _leaky(
                _dot(lat_sc[...], adj_sc[:, pl.ds(j * W, remw)])
            )

        @pl.when(i > 0)
        def _full_block():
            outT_ref[...] = _leaky(
                _dot(lat_sc[...], adj_sc[:, pl.ds(j * W, W)])
            )


@jax.jit
def kernel(adj, embeds):
    n, h = adj.shape
    d = embeds.shape[1]
    nblk = -(-n // W)
    rem = n - (nblk - 1) * W
    n128 = -(-n // 128) * 128
    remw = min(W, n128 - (nblk - 1) * W)

    embT = embeds.T
    body = functools.partial(
        _hgnn_body, nblk=nblk, rem=rem, remw=remw, h=h, d=d
    )
    retT = pl.pallas_call(
        body,
        grid=(2, nblk),
        in_specs=[
            pl.BlockSpec((W, h),
                         lambda p, i: (i * (1 - p) + (nblk - 1) * p, 0)),
            pl.BlockSpec((d, W),
                         lambda p, i: (0, i * (1 - p) + (nblk - 1) * p)),
        ],
        out_specs=pl.BlockSpec((d, W), lambda p, i: (0, nblk - 1 - i * p)),
        out_shape=jax.ShapeDtypeStruct((d, n), jnp.float32),
        scratch_shapes=[
            pltpu.VMEM((h, (nblk - 1) * W + remw), jnp.bfloat16),
            pltpu.VMEM((d, h), jnp.float32),
            pltpu.VMEM((d, h), jnp.bfloat16),
        ],
    )(adj, embT)
    return retT.T
